# speedup vs baseline: 1.1137x; 1.1137x over previous
import jax
import jax.numpy as jnp
from jax.experimental import pallas as pl

F = 126
G = 64


def _mlp_kernel(g_ref, w1, b1, w2, b2, w3, b3, w4, b4, out_ref):
    g = g_ref[...]
    z = jnp.maximum(jnp.dot(g, w1[...], preferred_element_type=jnp.float32) + b1[...], 0.0)
    z = jnp.maximum(jnp.dot(z, w2[...], preferred_element_type=jnp.float32) + b2[...], 0.0)
    z = jnp.maximum(jnp.dot(z, w3[...], preferred_element_type=jnp.float32) + b3[...], 0.0)
    out_ref[...] = jnp.dot(z, w4[...], preferred_element_type=jnp.float32) + b4[...]


def kernel(x, edge_index, edge_attr, batch, params):
    src = edge_index[0].astype(jnp.int32)
    dst = edge_index[1].astype(jnp.int32)
    attr = edge_attr.astype(jnp.int32)
    n = x.shape[0]
    h = x @ params["emb1_W"] + params["emb1_b"]
    cnt = jax.ops.segment_sum(jnp.ones((src.shape[0],), jnp.float32), dst, num_segments=n)
    avg_log = jnp.mean(jnp.log(cnt + 1.0))
    cnt_c = jnp.maximum(cnt, 1.0)[:, None]
    has = (cnt > 0)[:, None]
    amp = (jnp.log(cnt + 1.0) / avg_log)[:, None]
    att = (avg_log / jnp.log(jnp.maximum(cnt, 1.0) + 1.0))[:, None]
    for c in params["convs"]:
        A = h @ c["Wpre"][:F]
        B = h @ c["Wpre"][F:2 * F]
        Ctab = (params["edg_emb"] @ c["We"] + c["be"]) @ c["Wpre"][2 * F:] + c["bpre"]
        q = B[src] + Ctab[attr]
        Sq = jax.ops.segment_sum(q, dst, num_segments=n)
        Sq2 = jax.ops.segment_sum(q * q, dst, num_segments=n)
        Mn = jax.ops.segment_min(q, dst, num_segments=n)
        Mx = jax.ops.segment_max(q, dst, num_segments=n)
        mean = (cnt[:, None] * A + Sq) / cnt_c
        sqm = (cnt[:, None] * A * A + 2.0 * A * Sq + Sq2) / cnt_c
        std = jnp.sqrt(jax.nn.relu(sqm - mean * mean) + 1e-5)
        mn = jnp.where(has, A + Mn, 0.0)
        mx = jnp.where(has, A + Mx, 0.0)
        agg = jnp.concatenate([mean, mn, mx, std], axis=-1)
        Wf = c["Wpost"] @ c["Wlin"]
        bf = c["bpost"] @ c["Wlin"] + c["blin"]
        y = jnp.concatenate([h, agg, agg * amp, agg * att], axis=-1) @ Wf + bf
        m = jnp.mean(y, axis=0)
        v = jnp.var(y, axis=0)
        h = jax.nn.relu((y - m) / jnp.sqrt(v + 1e-5) * c["bn_g"] + c["bn_b"])
    g = jax.ops.segment_sum(h, batch, num_segments=G)
    (w1, b1), (w2, b2), (w3, b3), (w4, b4) = params["mlp"]
    out = pl.pallas_call(
        _mlp_kernel,
        out_shape=jax.ShapeDtypeStruct((G, 1), jnp.float32),
    )(g, w1, b1, w2, b2, w3, b3, w4, b4)
    return out


# speedup vs baseline: 1.6748x; 1.5038x over previous
import functools

import numpy as np

import jax
import jax.numpy as jnp
from jax import lax
from jax.experimental import pallas as pl
from jax.experimental.pallas import tpu as pltpu
from jax.experimental.pallas import tpu_sc as plsc

F = 126
FP = 128
N = 10000
NP = 10240
E = 160000
NG = 64

NC = 2
NS = 16
NW = NC * NS
EW = 2000
NWIN = E // EW
GPW = EW // 16
NPT = NP // NW
FLT_MAX = 3.4028235e38

NB = 2048
NB2 = 1024


def _sc_edge_body(bt, ctab, srch, dsth, attrh,
                  sq_o, sq2_o, mn_o, mx_o, cnt_o,
                  brow0, brow1, ct0, ct1, srcw, dstw, attrw, tmpw, cntacc,
                  asq0, asq20, amn0, amx0, asq1, asq21, amn1, amx1):
    wid = lax.axis_index("s") * NC + lax.axis_index("c")
    lanes = lax.broadcasted_iota(jnp.int32, (16,), 0)
    ones = jnp.full((16,), 1.0, jnp.float32)
    zero16 = jnp.zeros((16,), jnp.float32)
    big16 = jnp.full((16,), FLT_MAX, jnp.float32)

    for p in range(2):
        f0 = p * 64 + wid * 2
        pltpu.sync_copy(bt.at[f0], brow0)
        pltpu.sync_copy(bt.at[f0 + 1], brow1)
        pltpu.sync_copy(ctab.at[f0], ct0)
        pltpu.sync_copy(ctab.at[f0 + 1], ct1)

        def init_body(k, _, p=p):
            sl = pl.ds(k * 16, 16)
            asq0[sl] = zero16
            asq20[sl] = zero16
            amn0[sl] = big16
            amx0[sl] = -big16
            asq1[sl] = zero16
            asq21[sl] = zero16
            amn1[sl] = big16
            amx1[sl] = -big16
            if p == 0:
                cntacc[sl] = zero16
            return jnp.int32(0)

        lax.fori_loop(jnp.int32(0), jnp.int32(NP // 16), init_body, jnp.int32(0))

        def win_body(w, _, p=p):
            off = w * EW
            pltpu.sync_copy(srch.at[pl.ds(off, EW)], srcw)
            pltpu.sync_copy(dsth.at[pl.ds(off, EW)], dstw)
            pltpu.sync_copy(attrh.at[pl.ds(off, EW)], attrw)

            def grp_body(g, _):
                sl = pl.ds(g * 16, 16)
                srcv = srcw[sl]
                dstv = dstw[sl]
                attrv = attrw[sl]

                def cond(rem_i):
                    return jnp.sum(rem_i, dtype=jnp.int32) > 0

                def body(rem_i):
                    rem = rem_i > 0
                    plsc.store_scatter(tmpw, [dstv], lanes, mask=rem)
                    r = plsc.load_gather(tmpw, [dstv], mask=rem)
                    win = jnp.logical_and(rem, r == lanes)
                    if p == 0:
                        c = plsc.load_gather(cntacc, [dstv], mask=win)
                        plsc.store_scatter(cntacc, [dstv], c + ones, mask=win)
                    for brow, ct, asq, asq2, amn, amx in (
                            (brow0, ct0, asq0, asq20, amn0, amx0),
                            (brow1, ct1, asq1, asq21, amn1, amx1)):
                        b = plsc.load_gather(brow, [srcv], mask=win)
                        cg = plsc.load_gather(ct, [attrv], mask=win)
                        q = b + cg
                        s = plsc.load_gather(asq, [dstv], mask=win)
                        plsc.store_scatter(asq, [dstv], s + q, mask=win)
                        s2 = plsc.load_gather(asq2, [dstv], mask=win)
                        plsc.store_scatter(asq2, [dstv], s2 + q * q, mask=win)
                        mv = plsc.load_gather(amn, [dstv], mask=win)
                        plsc.store_scatter(amn, [dstv], jnp.minimum(mv, q), mask=win)
                        Mv = plsc.load_gather(amx, [dstv], mask=win)
                        plsc.store_scatter(amx, [dstv], jnp.maximum(Mv, q), mask=win)
                    return jnp.logical_and(
                        rem, jnp.logical_not(win)).astype(jnp.int32)

                lax.while_loop(cond, body, jnp.ones((16,), jnp.int32))
                return jnp.int32(0)

            lax.fori_loop(jnp.int32(0), jnp.int32(GPW), grp_body, jnp.int32(0))
            return jnp.int32(0)

        lax.fori_loop(jnp.int32(0), jnp.int32(NWIN), win_body, jnp.int32(0))

        pltpu.sync_copy(asq0, sq_o.at[f0])
        pltpu.sync_copy(asq1, sq_o.at[f0 + 1])
        pltpu.sync_copy(asq20, sq2_o.at[f0])
        pltpu.sync_copy(asq21, sq2_o.at[f0 + 1])
        pltpu.sync_copy(amn0, mn_o.at[f0])
        pltpu.sync_copy(amn1, mn_o.at[f0 + 1])
        pltpu.sync_copy(amx0, mx_o.at[f0])
        pltpu.sync_copy(amx1, mx_o.at[f0 + 1])
        if p == 0:
            s0 = wid * NPT
            pltpu.sync_copy(cntacc.at[pl.ds(s0, NPT)], cnt_o.at[pl.ds(s0, NPT)])


@functools.cache
def _get_sc_edge():
    return pl.kernel(
        _sc_edge_body,
        out_type=[jax.ShapeDtypeStruct((FP, NP), jnp.float32),
                  jax.ShapeDtypeStruct((FP, NP), jnp.float32),
                  jax.ShapeDtypeStruct((FP, NP), jnp.float32),
                  jax.ShapeDtypeStruct((FP, NP), jnp.float32),
                  jax.ShapeDtypeStruct((NP,), jnp.float32)],
        mesh=plsc.VectorSubcoreMesh(core_axis_name="c", subcore_axis_name="s",
                                    num_cores=NC, num_subcores=NS),
        compiler_params=pltpu.CompilerParams(needs_layout_passes=False),
        scratch_types=[pltpu.VMEM((NP,), jnp.float32),
                       pltpu.VMEM((NP,), jnp.float32),
                       pltpu.VMEM((16,), jnp.float32),
                       pltpu.VMEM((16,), jnp.float32),
                       pltpu.VMEM((EW,), jnp.int32),
                       pltpu.VMEM((EW,), jnp.int32),
                       pltpu.VMEM((EW,), jnp.int32),
                       pltpu.VMEM((NP,), jnp.int32),
                       pltpu.VMEM((NP,), jnp.float32),
                       pltpu.VMEM((NP,), jnp.float32),
                       pltpu.VMEM((NP,), jnp.float32),
                       pltpu.VMEM((NP,), jnp.float32),
                       pltpu.VMEM((NP,), jnp.float32),
                       pltpu.VMEM((NP,), jnp.float32),
                       pltpu.VMEM((NP,), jnp.float32),
                       pltpu.VMEM((NP,), jnp.float32),
                       pltpu.VMEM((NP,), jnp.float32)],
    )


def _dot(a, b):
    return jnp.dot(a, b, preferred_element_type=jnp.float32,
                   precision=lax.Precision.HIGHEST)


def _emb_body(xt_ref, w_ref, b_ref, o_ref):
    o_ref[...] = _dot(w_ref[...], xt_ref[...]) + b_ref[...]


def _ab_first_body(h_ref, wa_ref, wb_ref, a_ref, b_ref):
    h = h_ref[...]
    a_ref[...] = _dot(wa_ref[...], h)
    b_ref[...] = _dot(wb_ref[...], h)


def _ab_bn_body(y_ref, s_ref, g_ref, bb_ref, wa_ref, wb_ref,
                h_ref, a_ref, b_ref):
    m = s_ref[:, 0:1] * (1.0 / N)
    var = s_ref[:, 1:2] * (1.0 / N) - m * m
    h = jnp.maximum((y_ref[...] - m) * lax.rsqrt(var + 1e-5) * g_ref[...]
                    + bb_ref[...], 0.0)
    h_ref[...] = h
    a_ref[...] = _dot(wa_ref[...], h)
    b_ref[...] = _dot(wb_ref[...], h)


def _avg_body(c_ref, o_ref):
    o_ref[0, 0] = jnp.sum(jnp.log(c_ref[...] + 1.0)) * (1.0 / N)


def _post_body(h_ref, a_ref, sq_ref, sq2_ref, mn_ref, mx_ref, cnt_ref,
               avg_ref, wf_ref, bf_ref, y_ref, s_out, acc):
    i = pl.program_id(0)
    cntr = cnt_ref[0]
    cntc = jnp.maximum(cntr, 1.0)
    has = cntr > 0.0
    al = avg_ref[0, 0]
    amp = jnp.log(cntr + 1.0) * (1.0 / al)
    att = al / jnp.log(cntc + 1.0)
    A = a_ref[...]
    Sq = sq_ref[...]
    Sq2 = sq2_ref[...]
    mean = (cntr * A + Sq) / cntc
    sqm = (cntr * A * A + 2.0 * A * Sq + Sq2) / cntc
    std = jnp.sqrt(jnp.maximum(sqm - mean * mean, 0.0) + 1e-5)
    mn = jnp.where(has, A + mn_ref[...], 0.0)
    mx = jnp.where(has, A + mx_ref[...], 0.0)
    agg = jnp.concatenate([mean, mn, mx, std], axis=0)
    stacked = jnp.concatenate([h_ref[...], agg, agg * amp, agg * att], axis=0)
    y = _dot(wf_ref[...], stacked) + bf_ref[...]
    y_ref[...] = y
    col = lax.broadcasted_iota(jnp.int32, (1, NB2), 1) + i * NB2
    ym = jnp.where(col < N, y, 0.0)

    @pl.when(i == 0)
    def _():
        acc[...] = jnp.zeros((FP, 128), jnp.float32)

    acc[:, 0:1] += jnp.sum(ym, axis=1, keepdims=True)
    acc[:, 1:2] += jnp.sum(ym * ym, axis=1, keepdims=True)

    @pl.when(i == pl.num_programs(0) - 1)
    def _():
        s_out[...] = acc[...]


def _pool_body(y_ref, s_ref, g_ref, bb_ref, batch_ref,
               w1, b1, w2, b2, w3, b3, w4, b4, o_ref, acc):
    i = pl.program_id(0)
    m = s_ref[:, 0:1] * (1.0 / N)
    var = s_ref[:, 1:2] * (1.0 / N) - m * m
    h = jnp.maximum((y_ref[...] - m) * lax.rsqrt(var + 1e-5) * g_ref[...]
                    + bb_ref[...], 0.0)
    bb = batch_ref[0]
    gi = lax.broadcasted_iota(jnp.int32, (128, 1), 0)
    oh = (gi == bb).astype(jnp.float32)
    part = lax.dot_general(h, oh, (((1,), (1,)), ((), ())),
                           preferred_element_type=jnp.float32,
                           precision=lax.Precision.HIGHEST)

    @pl.when(i == 0)
    def _():
        acc[...] = jnp.zeros((FP, 128), jnp.float32)

    acc[...] += part

    @pl.when(i == pl.num_programs(0) - 1)
    def _():
        gt = acc[...]
        z = jnp.maximum(_dot(w1[...], gt) + b1[...], 0.0)
        z = jnp.maximum(_dot(w2[...], z) + b2[...], 0.0)
        z = jnp.maximum(_dot(w3[...], z) + b3[...], 0.0)
        o_ref[...] = _dot(w4[...], z) + b4[...]


_I0 = np.int32(0)


def _full(shape):
    return pl.BlockSpec(shape, lambda i: tuple(_I0 for _ in shape))


def _nodes(rows, nb):
    return pl.BlockSpec((rows, nb), lambda i: (_I0, i))


def _build_tc(interpret=False):
    f32 = jnp.float32
    emb = pl.pallas_call(
        _emb_body, grid=(NP // NB,), interpret=interpret,
        in_specs=[_nodes(8, NB), _full((FP, 8)), _full((FP, 1))],
        out_specs=_nodes(FP, NB),
        out_shape=jax.ShapeDtypeStruct((FP, NP), f32))
    ab_first = pl.pallas_call(
        _ab_first_body, grid=(NP // NB,), interpret=interpret,
        in_specs=[_nodes(FP, NB), _full((FP, FP)), _full((FP, FP))],
        out_specs=[_nodes(FP, NB), _nodes(FP, NB)],
        out_shape=[jax.ShapeDtypeStruct((FP, NP), f32)] * 2)
    ab_bn = pl.pallas_call(
        _ab_bn_body, grid=(NP // NB,), interpret=interpret,
        in_specs=[_nodes(FP, NB), _full((FP, 128)), _full((FP, 1)),
                  _full((FP, 1)), _full((FP, FP)), _full((FP, FP))],
        out_specs=[_nodes(FP, NB)] * 3,
        out_shape=[jax.ShapeDtypeStruct((FP, NP), f32)] * 3)
    avg = pl.pallas_call(
        _avg_body, interpret=interpret,
        in_specs=[pl.BlockSpec((NP // 128, 128), lambda: (_I0, _I0))],
        out_specs=pl.BlockSpec(memory_space=pltpu.SMEM),
        out_shape=jax.ShapeDtypeStruct((1, 1), f32))
    post = pl.pallas_call(
        _post_body, grid=(NP // NB2,), interpret=interpret,
        in_specs=[_nodes(FP, NB2), _nodes(FP, NB2), _nodes(FP, NB2),
                  _nodes(FP, NB2), _nodes(FP, NB2), _nodes(FP, NB2),
                  pl.BlockSpec((1, 1, NB2), lambda i: (i, _I0, _I0)),
                  _full((1, 1)), _full((FP, 13 * FP)), _full((FP, 1))],
        out_specs=[_nodes(FP, NB2), _full((FP, 128))],
        out_shape=[jax.ShapeDtypeStruct((FP, NP), f32),
                   jax.ShapeDtypeStruct((FP, 128), f32)],
        scratch_shapes=[pltpu.VMEM((FP, 128), f32)])
    pool = pl.pallas_call(
        _pool_body, grid=(NP // NB,), interpret=interpret,
        in_specs=[_nodes(FP, NB), _full((FP, 128)), _full((FP, 1)),
                  _full((FP, 1)),
                  pl.BlockSpec((1, 1, NB), lambda i: (i, _I0, _I0))]
                 + [_full((FP, FP)), _full((FP, 1))] * 4,
        out_specs=_full((FP, 128)),
        out_shape=jax.ShapeDtypeStruct((FP, 128), f32),
        scratch_shapes=[pltpu.VMEM((FP, 128), f32)])
    return emb, ab_first, ab_bn, avg, post, pool


_emb, _ab_first, _ab_bn, _avg, _post, _pool = _build_tc()


def _padt(w):
    return jnp.zeros((FP, FP), jnp.float32).at[:w.shape[1], :w.shape[0]].set(w.T)


def _padc(b, rows=FP):
    return jnp.zeros((rows, 1), jnp.float32).at[:b.shape[0], 0].set(b)


def _forward(x, edge_index, edge_attr, batch, params, tc, sc):
    _emb, _ab_first, _ab_bn, _avg, _post, _pool = tc
    src = edge_index[0].astype(jnp.int32)
    dst = edge_index[1].astype(jnp.int32)
    attr = edge_attr.astype(jnp.int32)

    xt = jnp.zeros((8, NP), jnp.float32).at[:5, :N].set(x.T)
    batch3 = (jnp.full((NP,), 127, jnp.int32).at[:N].set(batch.astype(jnp.int32))
              .reshape(NP // NB, 1, NB))

    embw = (jnp.zeros((FP, 8), jnp.float32)
            .at[:F, :5].set(params["emb1_W"].T))
    h = _emb(xt, embw, _padc(params["emb1_b"]))

    y = bns = cnt3 = avg = None
    for l, c in enumerate(params["convs"]):
        wat = _padt(c["Wpre"][:F])
        wbt = _padt(c["Wpre"][F:2 * F])
        ctab = ((params["edg_emb"] @ c["We"] + c["be"]) @ c["Wpre"][2 * F:]
                + c["bpre"])
        ctabt = jnp.zeros((FP, 16), jnp.float32).at[:F, :10].set(ctab.T)
        wf = c["Wpost"] @ c["Wlin"]
        wft = jnp.concatenate(
            [_padt(wf[k * F:(k + 1) * F]) for k in range(13)], axis=1)
        bf = _padc(c["bpost"] @ c["Wlin"] + c["blin"])

        if l == 0:
            at, bt = _ab_first(h, wat, wbt)
        else:
            cp = params["convs"][l - 1]
            h, at, bt = _ab_bn(y, bns, _padc(cp["bn_g"]), _padc(cp["bn_b"]),
                               wat, wbt)
        sq, sq2, mn, mx, cnt = sc(bt, ctabt, src, dst, attr)
        if l == 0:
            avg = _avg(cnt.reshape(NP // 128, 128))
            cnt3 = cnt.reshape(NP // NB2, 1, NB2)
        y, bns = _post(h, at, sq, sq2, mn, mx, cnt3, avg, wft, bf)

    c3 = params["convs"][3]
    (w1, b1), (w2, b2), (w3, b3), (w4, b4) = params["mlp"]
    out = _pool(y, bns, _padc(c3["bn_g"]), _padc(c3["bn_b"]), batch3,
                _padt(w1), _padc(b1), _padt(w2), _padc(b2),
                _padt(w3), _padc(b3), _padt(w4), _padc(b4))
    return out[0, :NG][:, None]


def kernel(x, edge_index, edge_attr, batch, params):
    return _forward(x, edge_index, edge_attr, batch, params,
                    (_emb, _ab_first, _ab_bn, _avg, _post, _pool),
                    lambda *a: _get_sc_edge()(*a))


# speedup vs baseline: 1.7361x; 1.0366x over previous
import functools

import numpy as np

import jax
import jax.numpy as jnp
from jax import lax
from jax.experimental import pallas as pl
from jax.experimental.pallas import tpu as pltpu
from jax.experimental.pallas import tpu_sc as plsc

F = 126
FP = 128
N = 10000
NP = 10240
E = 160000
NG = 64

NC = 2
NS = 16
NW = NC * NS
EW = 2000
NWIN = E // EW
GPW = EW // 16
GUNROLL = 5
NPT = NP // NW
FLT_MAX = 3.4028235e38

NB = 2048
NB2 = 1024


def _sc_edge_body(bt, ctab, srch, dsth, attrh,
                  sq_o, sq2_o, mn_o, mx_o, cnt_o,
                  brow0, brow1, ct0, ct1, srcw, dstw, attrw, tmpw, cntacc,
                  asq0, asq20, amn0, amx0, asq1, asq21, amn1, amx1):
    wid = lax.axis_index("s") * NC + lax.axis_index("c")
    lanes = lax.broadcasted_iota(jnp.int32, (16,), 0)
    ones = jnp.full((16,), 1.0, jnp.float32)
    zero16 = jnp.zeros((16,), jnp.float32)
    big16 = jnp.full((16,), FLT_MAX, jnp.float32)

    for p in range(2):
        f0 = p * 64 + wid * 2
        pltpu.sync_copy(bt.at[f0], brow0)
        pltpu.sync_copy(bt.at[f0 + 1], brow1)
        pltpu.sync_copy(ctab.at[f0], ct0)
        pltpu.sync_copy(ctab.at[f0 + 1], ct1)

        def init_body(k, _, p=p):
            sl = pl.ds(k * 16, 16)
            asq0[sl] = zero16
            asq20[sl] = zero16
            amn0[sl] = big16
            amx0[sl] = -big16
            asq1[sl] = zero16
            asq21[sl] = zero16
            amn1[sl] = big16
            amx1[sl] = -big16
            if p == 0:
                cntacc[sl] = zero16
            return jnp.int32(0)

        lax.fori_loop(jnp.int32(0), jnp.int32(NP // 16), init_body, jnp.int32(0))

        def win_body(w, _, p=p):
            off = w * EW
            pltpu.sync_copy(srch.at[pl.ds(off, EW)], srcw)
            pltpu.sync_copy(dsth.at[pl.ds(off, EW)], dstw)
            pltpu.sync_copy(attrh.at[pl.ds(off, EW)], attrw)

            def do_round(srcv, dstv, attrv, remb, p=p):
                plsc.store_scatter(tmpw, [dstv], lanes, mask=remb)
                r = plsc.load_gather(tmpw, [dstv], mask=remb)
                if remb is None:
                    win = r == lanes
                    left = r != lanes
                else:
                    win = jnp.logical_and(remb, r == lanes)
                    left = jnp.logical_and(remb, jnp.logical_not(win))
                if p == 0:
                    c = plsc.load_gather(cntacc, [dstv], mask=win)
                    plsc.store_scatter(cntacc, [dstv], c + ones, mask=win)
                for brow, ct, asq, asq2, amn, amx in (
                        (brow0, ct0, asq0, asq20, amn0, amx0),
                        (brow1, ct1, asq1, asq21, amn1, amx1)):
                    b = plsc.load_gather(brow, [srcv], mask=win)
                    cg = plsc.load_gather(ct, [attrv], mask=win)
                    q = b + cg
                    s = plsc.load_gather(asq, [dstv], mask=win)
                    plsc.store_scatter(asq, [dstv], s + q, mask=win)
                    s2 = plsc.load_gather(asq2, [dstv], mask=win)
                    plsc.store_scatter(asq2, [dstv], s2 + q * q, mask=win)
                    mv = plsc.load_gather(amn, [dstv], mask=win)
                    plsc.store_scatter(amn, [dstv], jnp.minimum(mv, q), mask=win)
                    Mv = plsc.load_gather(amx, [dstv], mask=win)
                    plsc.store_scatter(amx, [dstv], jnp.maximum(Mv, q), mask=win)
                return left

            def grp_one(g):
                sl = pl.ds(g * 16, 16)
                srcv = srcw[sl]
                dstv = dstw[sl]
                attrv = attrw[sl]
                left = do_round(srcv, dstv, attrv, None)
                nleft = plsc.all_reduce_population_count(left)[0]

                @pl.when(nleft > 0)
                def _():
                    def cond(carry):
                        return carry[1] > 0

                    def wbody(carry):
                        rem_i, _ = carry
                        l2 = do_round(srcv, dstv, attrv, rem_i > 0)
                        return (l2.astype(jnp.int32),
                                plsc.all_reduce_population_count(l2)[0])

                    lax.while_loop(cond, wbody,
                                   (left.astype(jnp.int32), nleft))

            def grp_body(gg, _):
                g0 = gg * jnp.int32(GUNROLL)
                for u in range(GUNROLL):
                    grp_one(g0 + jnp.int32(u))
                return jnp.int32(0)

            lax.fori_loop(jnp.int32(0), jnp.int32(GPW // GUNROLL), grp_body,
                          jnp.int32(0))
            return jnp.int32(0)

        lax.fori_loop(jnp.int32(0), jnp.int32(NWIN), win_body, jnp.int32(0))

        pltpu.sync_copy(asq0, sq_o.at[f0])
        pltpu.sync_copy(asq1, sq_o.at[f0 + 1])
        pltpu.sync_copy(asq20, sq2_o.at[f0])
        pltpu.sync_copy(asq21, sq2_o.at[f0 + 1])
        pltpu.sync_copy(amn0, mn_o.at[f0])
        pltpu.sync_copy(amn1, mn_o.at[f0 + 1])
        pltpu.sync_copy(amx0, mx_o.at[f0])
        pltpu.sync_copy(amx1, mx_o.at[f0 + 1])
        if p == 0:
            s0 = wid * NPT
            pltpu.sync_copy(cntacc.at[pl.ds(s0, NPT)], cnt_o.at[pl.ds(s0, NPT)])


@functools.cache
def _get_sc_edge():
    return pl.kernel(
        _sc_edge_body,
        out_type=[jax.ShapeDtypeStruct((FP, NP), jnp.float32),
                  jax.ShapeDtypeStruct((FP, NP), jnp.float32),
                  jax.ShapeDtypeStruct((FP, NP), jnp.float32),
                  jax.ShapeDtypeStruct((FP, NP), jnp.float32),
                  jax.ShapeDtypeStruct((NP,), jnp.float32)],
        mesh=plsc.VectorSubcoreMesh(core_axis_name="c", subcore_axis_name="s",
                                    num_cores=NC, num_subcores=NS),
        compiler_params=pltpu.CompilerParams(needs_layout_passes=False),
        scratch_types=[pltpu.VMEM((NP,), jnp.float32),
                       pltpu.VMEM((NP,), jnp.float32),
                       pltpu.VMEM((16,), jnp.float32),
                       pltpu.VMEM((16,), jnp.float32),
                       pltpu.VMEM((EW,), jnp.int32),
                       pltpu.VMEM((EW,), jnp.int32),
                       pltpu.VMEM((EW,), jnp.int32),
                       pltpu.VMEM((NP,), jnp.int32),
                       pltpu.VMEM((NP,), jnp.float32),
                       pltpu.VMEM((NP,), jnp.float32),
                       pltpu.VMEM((NP,), jnp.float32),
                       pltpu.VMEM((NP,), jnp.float32),
                       pltpu.VMEM((NP,), jnp.float32),
                       pltpu.VMEM((NP,), jnp.float32),
                       pltpu.VMEM((NP,), jnp.float32),
                       pltpu.VMEM((NP,), jnp.float32),
                       pltpu.VMEM((NP,), jnp.float32)],
    )


def _dot(a, b):
    return jnp.dot(a, b, preferred_element_type=jnp.float32,
                   precision=lax.Precision.HIGHEST)


def _emb_body(xt_ref, w_ref, b_ref, o_ref):
    o_ref[...] = _dot(w_ref[...], xt_ref[...]) + b_ref[...]


def _ab_first_body(h_ref, wa_ref, wb_ref, a_ref, b_ref):
    h = h_ref[...]
    a_ref[...] = _dot(wa_ref[...], h)
    b_ref[...] = _dot(wb_ref[...], h)


def _ab_bn_body(y_ref, s_ref, g_ref, bb_ref, wa_ref, wb_ref,
                h_ref, a_ref, b_ref):
    m = s_ref[:, 0:1] * (1.0 / N)
    var = s_ref[:, 1:2] * (1.0 / N) - m * m
    h = jnp.maximum((y_ref[...] - m) * lax.rsqrt(var + 1e-5) * g_ref[...]
                    + bb_ref[...], 0.0)
    h_ref[...] = h
    a_ref[...] = _dot(wa_ref[...], h)
    b_ref[...] = _dot(wb_ref[...], h)


def _avg_body(c_ref, o_ref):
    o_ref[0, 0] = jnp.sum(jnp.log(c_ref[...] + 1.0)) * (1.0 / N)


def _post_body(h_ref, a_ref, sq_ref, sq2_ref, mn_ref, mx_ref, cnt_ref,
               avg_ref, wf_ref, bf_ref, y_ref, s_out, acc):
    i = pl.program_id(0)
    cntr = cnt_ref[0]
    cntc = jnp.maximum(cntr, 1.0)
    has = cntr > 0.0
    al = avg_ref[0, 0]
    amp = jnp.log(cntr + 1.0) * (1.0 / al)
    att = al / jnp.log(cntc + 1.0)
    A = a_ref[...]
    Sq = sq_ref[...]
    Sq2 = sq2_ref[...]
    mean = (cntr * A + Sq) / cntc
    sqm = (cntr * A * A + 2.0 * A * Sq + Sq2) / cntc
    std = jnp.sqrt(jnp.maximum(sqm - mean * mean, 0.0) + 1e-5)
    mn = jnp.where(has, A + mn_ref[...], 0.0)
    mx = jnp.where(has, A + mx_ref[...], 0.0)
    agg = jnp.concatenate([mean, mn, mx, std], axis=0)
    stacked = jnp.concatenate([h_ref[...], agg, agg * amp, agg * att], axis=0)
    y = _dot(wf_ref[...], stacked) + bf_ref[...]
    y_ref[...] = y
    col = lax.broadcasted_iota(jnp.int32, (1, NB2), 1) + i * NB2
    ym = jnp.where(col < N, y, 0.0)

    @pl.when(i == 0)
    def _():
        acc[...] = jnp.zeros((FP, 128), jnp.float32)

    acc[:, 0:1] += jnp.sum(ym, axis=1, keepdims=True)
    acc[:, 1:2] += jnp.sum(ym * ym, axis=1, keepdims=True)

    @pl.when(i == pl.num_programs(0) - 1)
    def _():
        s_out[...] = acc[...]


def _pool_body(y_ref, s_ref, g_ref, bb_ref, batch_ref,
               w1, b1, w2, b2, w3, b3, w4, b4, o_ref, acc):
    i = pl.program_id(0)
    m = s_ref[:, 0:1] * (1.0 / N)
    var = s_ref[:, 1:2] * (1.0 / N) - m * m
    h = jnp.maximum((y_ref[...] - m) * lax.rsqrt(var + 1e-5) * g_ref[...]
                    + bb_ref[...], 0.0)
    bb = batch_ref[0]
    gi = lax.broadcasted_iota(jnp.int32, (128, 1), 0)
    oh = (gi == bb).astype(jnp.float32)
    part = lax.dot_general(h, oh, (((1,), (1,)), ((), ())),
                           preferred_element_type=jnp.float32,
                           precision=lax.Precision.HIGHEST)

    @pl.when(i == 0)
    def _():
        acc[...] = jnp.zeros((FP, 128), jnp.float32)

    acc[...] += part

    @pl.when(i == pl.num_programs(0) - 1)
    def _():
        gt = acc[...]
        z = jnp.maximum(_dot(w1[...], gt) + b1[...], 0.0)
        z = jnp.maximum(_dot(w2[...], z) + b2[...], 0.0)
        z = jnp.maximum(_dot(w3[...], z) + b3[...], 0.0)
        o_ref[...] = _dot(w4[...], z) + b4[...]


_I0 = np.int32(0)


def _full(shape):
    return pl.BlockSpec(shape, lambda i: tuple(_I0 for _ in shape))


def _nodes(rows, nb):
    return pl.BlockSpec((rows, nb), lambda i: (_I0, i))


def _build_tc(interpret=False):
    f32 = jnp.float32
    emb = pl.pallas_call(
        _emb_body, grid=(NP // NB,), interpret=interpret,
        in_specs=[_nodes(8, NB), _full((FP, 8)), _full((FP, 1))],
        out_specs=_nodes(FP, NB),
        out_shape=jax.ShapeDtypeStruct((FP, NP), f32))
    ab_first = pl.pallas_call(
        _ab_first_body, grid=(NP // NB,), interpret=interpret,
        in_specs=[_nodes(FP, NB), _full((FP, FP)), _full((FP, FP))],
        out_specs=[_nodes(FP, NB), _nodes(FP, NB)],
        out_shape=[jax.ShapeDtypeStruct((FP, NP), f32)] * 2)
    ab_bn = pl.pallas_call(
        _ab_bn_body, grid=(NP // NB,), interpret=interpret,
        in_specs=[_nodes(FP, NB), _full((FP, 128)), _full((FP, 1)),
                  _full((FP, 1)), _full((FP, FP)), _full((FP, FP))],
        out_specs=[_nodes(FP, NB)] * 3,
        out_shape=[jax.ShapeDtypeStruct((FP, NP), f32)] * 3)
    avg = pl.pallas_call(
        _avg_body, interpret=interpret,
        in_specs=[pl.BlockSpec((NP // 128, 128), lambda: (_I0, _I0))],
        out_specs=pl.BlockSpec(memory_space=pltpu.SMEM),
        out_shape=jax.ShapeDtypeStruct((1, 1), f32))
    post = pl.pallas_call(
        _post_body, grid=(NP // NB2,), interpret=interpret,
        in_specs=[_nodes(FP, NB2), _nodes(FP, NB2), _nodes(FP, NB2),
                  _nodes(FP, NB2), _nodes(FP, NB2), _nodes(FP, NB2),
                  pl.BlockSpec((1, 1, NB2), lambda i: (i, _I0, _I0)),
                  _full((1, 1)), _full((FP, 13 * FP)), _full((FP, 1))],
        out_specs=[_nodes(FP, NB2), _full((FP, 128))],
        out_shape=[jax.ShapeDtypeStruct((FP, NP), f32),
                   jax.ShapeDtypeStruct((FP, 128), f32)],
        scratch_shapes=[pltpu.VMEM((FP, 128), f32)])
    pool = pl.pallas_call(
        _pool_body, grid=(NP // NB,), interpret=interpret,
        in_specs=[_nodes(FP, NB), _full((FP, 128)), _full((FP, 1)),
                  _full((FP, 1)),
                  pl.BlockSpec((1, 1, NB), lambda i: (i, _I0, _I0))]
                 + [_full((FP, FP)), _full((FP, 1))] * 4,
        out_specs=_full((FP, 128)),
        out_shape=jax.ShapeDtypeStruct((FP, 128), f32),
        scratch_shapes=[pltpu.VMEM((FP, 128), f32)])
    return emb, ab_first, ab_bn, avg, post, pool


_emb, _ab_first, _ab_bn, _avg, _post, _pool = _build_tc()


def _padt(w):
    return jnp.zeros((FP, FP), jnp.float32).at[:w.shape[1], :w.shape[0]].set(w.T)


def _padc(b, rows=FP):
    return jnp.zeros((rows, 1), jnp.float32).at[:b.shape[0], 0].set(b)


def _forward(x, edge_index, edge_attr, batch, params, tc, sc):
    _emb, _ab_first, _ab_bn, _avg, _post, _pool = tc
    src = edge_index[0].astype(jnp.int32)
    dst = edge_index[1].astype(jnp.int32)
    attr = edge_attr.astype(jnp.int32)

    xt = jnp.zeros((8, NP), jnp.float32).at[:5, :N].set(x.T)
    batch3 = (jnp.full((NP,), 127, jnp.int32).at[:N].set(batch.astype(jnp.int32))
              .reshape(NP // NB, 1, NB))

    embw = (jnp.zeros((FP, 8), jnp.float32)
            .at[:F, :5].set(params["emb1_W"].T))
    h = _emb(xt, embw, _padc(params["emb1_b"]))

    y = bns = cnt3 = avg = None
    for l, c in enumerate(params["convs"]):
        wat = _padt(c["Wpre"][:F])
        wbt = _padt(c["Wpre"][F:2 * F])
        ctab = ((params["edg_emb"] @ c["We"] + c["be"]) @ c["Wpre"][2 * F:]
                + c["bpre"])
        ctabt = jnp.zeros((FP, 16), jnp.float32).at[:F, :10].set(ctab.T)
        wf = c["Wpost"] @ c["Wlin"]
        wft = jnp.concatenate(
            [_padt(wf[k * F:(k + 1) * F]) for k in range(13)], axis=1)
        bf = _padc(c["bpost"] @ c["Wlin"] + c["blin"])

        if l == 0:
            at, bt = _ab_first(h, wat, wbt)
        else:
            cp = params["convs"][l - 1]
            h, at, bt = _ab_bn(y, bns, _padc(cp["bn_g"]), _padc(cp["bn_b"]),
                               wat, wbt)
        sq, sq2, mn, mx, cnt = sc(bt, ctabt, src, dst, attr)
        if l == 0:
            avg = _avg(cnt.reshape(NP // 128, 128))
            cnt3 = cnt.reshape(NP // NB2, 1, NB2)
        y, bns = _post(h, at, sq, sq2, mn, mx, cnt3, avg, wft, bf)

    c3 = params["convs"][3]
    (w1, b1), (w2, b2), (w3, b3), (w4, b4) = params["mlp"]
    out = _pool(y, bns, _padc(c3["bn_g"]), _padc(c3["bn_b"]), batch3,
                _padt(w1), _padc(b1), _padt(w2), _padc(b2),
                _padt(w3), _padc(b3), _padt(w4), _padc(b4))
    return out[0, :NG][:, None]


def kernel(x, edge_index, edge_attr, batch, params):
    return _forward(x, edge_index, edge_attr, batch, params,
                    (_emb, _ab_first, _ab_bn, _avg, _post, _pool),
                    lambda *a: _get_sc_edge()(*a))


# speedup vs baseline: 1.9875x; 1.1448x over previous
import functools

import numpy as np

import jax
import jax.numpy as jnp
from jax import lax
from jax.experimental import pallas as pl
from jax.experimental.pallas import tpu as pltpu
from jax.experimental.pallas import tpu_sc as plsc

F = 126
FP = 128
N = 10000
NP = 10240
E = 160000
NG = 64

NC = 2
NS = 16
NW = NC * NS
EW = 2000
NWIN = E // EW
GPW = EW // 16
GUNROLL = 5
NPT = NP // NW
FLT_MAX = 3.4028235e38

NB = 2048
NB2 = 1024


def _sc_edge_body(bt, ctab, srch, dsth, attrh,
                  sq_o, sq2_o, mn_o, mx_o, cnt_o,
                  brow0, brow1, ct0, ct1, srcw, dstw, attrw, tmpw, cntacc,
                  asq0, asq20, amn0, amx0, asq1, asq21, amn1, amx1):
    wid = lax.axis_index("s") * NC + lax.axis_index("c")
    lanes = lax.broadcasted_iota(jnp.int32, (16,), 0)
    ones = jnp.full((16,), 1.0, jnp.float32)
    zero16 = jnp.zeros((16,), jnp.float32)
    big16 = jnp.full((16,), FLT_MAX, jnp.float32)

    for p in range(2):
        f0 = p * 64 + wid * 2
        pltpu.sync_copy(bt.at[f0], brow0)
        pltpu.sync_copy(bt.at[f0 + 1], brow1)
        pltpu.sync_copy(ctab.at[f0], ct0)
        pltpu.sync_copy(ctab.at[f0 + 1], ct1)

        def init_body(k, _, p=p):
            sl = pl.ds(k * 16, 16)
            asq0[sl] = zero16
            asq20[sl] = zero16
            amn0[sl] = big16
            amx0[sl] = -big16
            asq1[sl] = zero16
            asq21[sl] = zero16
            amn1[sl] = big16
            amx1[sl] = -big16
            if p == 0:
                cntacc[sl] = zero16
            return jnp.int32(0)

        lax.fori_loop(jnp.int32(0), jnp.int32(NP // 16), init_body, jnp.int32(0))

        def win_body(w, _, p=p):
            off = w * EW
            pltpu.sync_copy(srch.at[pl.ds(off, EW)], srcw)
            pltpu.sync_copy(dsth.at[pl.ds(off, EW)], dstw)
            pltpu.sync_copy(attrh.at[pl.ds(off, EW)], attrw)

            def mm_round(q0, q1, dstv, remb):
                plsc.store_scatter(tmpw, [dstv], lanes, mask=remb)
                r = plsc.load_gather(tmpw, [dstv], mask=remb)
                if remb is None:
                    win = r == lanes
                    left = r != lanes
                else:
                    win = jnp.logical_and(remb, r == lanes)
                    left = jnp.logical_and(remb, jnp.logical_not(win))
                for q, amn, amx in ((q0, amn0, amx0), (q1, amn1, amx1)):
                    mv = plsc.load_gather(amn, [dstv], mask=win)
                    plsc.store_scatter(amn, [dstv], jnp.minimum(mv, q),
                                       mask=win)
                    Mv = plsc.load_gather(amx, [dstv], mask=win)
                    plsc.store_scatter(amx, [dstv], jnp.maximum(Mv, q),
                                       mask=win)
                return left

            def grp_one(g, p=p):
                sl = pl.ds(g * 16, 16)
                srcv = srcw[sl]
                dstv = dstw[sl]
                attrv = attrw[sl]
                if p == 0:
                    plsc.addupdate_scatter(cntacc, [dstv], ones)
                qs = []
                for brow, ct, asq, asq2 in (
                        (brow0, ct0, asq0, asq20),
                        (brow1, ct1, asq1, asq21)):
                    b = plsc.load_gather(brow, [srcv])
                    cg = plsc.load_gather(ct, [attrv])
                    q = b + cg
                    plsc.addupdate_scatter(asq, [dstv], q)
                    plsc.addupdate_scatter(asq2, [dstv], q * q)
                    qs.append(q)
                left = mm_round(qs[0], qs[1], dstv, None)
                nleft = plsc.all_reduce_population_count(left)[0]

                @pl.when(nleft > 0)
                def _():
                    def cond(carry):
                        return carry[1] > 0

                    def wbody(carry):
                        rem_i, _ = carry
                        l2 = mm_round(qs[0], qs[1], dstv, rem_i > 0)
                        return (l2.astype(jnp.int32),
                                plsc.all_reduce_population_count(l2)[0])

                    lax.while_loop(cond, wbody,
                                   (left.astype(jnp.int32), nleft))

            def grp_body(gg, _):
                g0 = gg * jnp.int32(GUNROLL)
                for u in range(GUNROLL):
                    grp_one(g0 + jnp.int32(u))
                return jnp.int32(0)

            lax.fori_loop(jnp.int32(0), jnp.int32(GPW // GUNROLL), grp_body,
                          jnp.int32(0))
            return jnp.int32(0)

        lax.fori_loop(jnp.int32(0), jnp.int32(NWIN), win_body, jnp.int32(0))

        pltpu.sync_copy(asq0, sq_o.at[f0])
        pltpu.sync_copy(asq1, sq_o.at[f0 + 1])
        pltpu.sync_copy(asq20, sq2_o.at[f0])
        pltpu.sync_copy(asq21, sq2_o.at[f0 + 1])
        pltpu.sync_copy(amn0, mn_o.at[f0])
        pltpu.sync_copy(amn1, mn_o.at[f0 + 1])
        pltpu.sync_copy(amx0, mx_o.at[f0])
        pltpu.sync_copy(amx1, mx_o.at[f0 + 1])
        if p == 0:
            s0 = wid * NPT
            pltpu.sync_copy(cntacc.at[pl.ds(s0, NPT)], cnt_o.at[pl.ds(s0, NPT)])


@functools.cache
def _get_sc_edge():
    return pl.kernel(
        _sc_edge_body,
        out_type=[jax.ShapeDtypeStruct((FP, NP), jnp.float32),
                  jax.ShapeDtypeStruct((FP, NP), jnp.float32),
                  jax.ShapeDtypeStruct((FP, NP), jnp.float32),
                  jax.ShapeDtypeStruct((FP, NP), jnp.float32),
                  jax.ShapeDtypeStruct((NP,), jnp.float32)],
        mesh=plsc.VectorSubcoreMesh(core_axis_name="c", subcore_axis_name="s",
                                    num_cores=NC, num_subcores=NS),
        compiler_params=pltpu.CompilerParams(needs_layout_passes=False),
        scratch_types=[pltpu.VMEM((NP,), jnp.float32),
                       pltpu.VMEM((NP,), jnp.float32),
                       pltpu.VMEM((16,), jnp.float32),
                       pltpu.VMEM((16,), jnp.float32),
                       pltpu.VMEM((EW,), jnp.int32),
                       pltpu.VMEM((EW,), jnp.int32),
                       pltpu.VMEM((EW,), jnp.int32),
                       pltpu.VMEM((NP,), jnp.int32),
                       pltpu.VMEM((NP,), jnp.float32),
                       pltpu.VMEM((NP,), jnp.float32),
                       pltpu.VMEM((NP,), jnp.float32),
                       pltpu.VMEM((NP,), jnp.float32),
                       pltpu.VMEM((NP,), jnp.float32),
                       pltpu.VMEM((NP,), jnp.float32),
                       pltpu.VMEM((NP,), jnp.float32),
                       pltpu.VMEM((NP,), jnp.float32),
                       pltpu.VMEM((NP,), jnp.float32)],
    )


def _dot(a, b):
    return jnp.dot(a, b, preferred_element_type=jnp.float32,
                   precision=lax.Precision.HIGHEST)


def _emb_body(xt_ref, w_ref, b_ref, o_ref):
    o_ref[...] = _dot(w_ref[...], xt_ref[...]) + b_ref[...]


def _ab_first_body(h_ref, wa_ref, wb_ref, a_ref, b_ref):
    h = h_ref[...]
    a_ref[...] = _dot(wa_ref[...], h)
    b_ref[...] = _dot(wb_ref[...], h)


def _ab_bn_body(y_ref, s_ref, g_ref, bb_ref, wa_ref, wb_ref,
                h_ref, a_ref, b_ref):
    m = s_ref[:, 0:1] * (1.0 / N)
    var = s_ref[:, 1:2] * (1.0 / N) - m * m
    h = jnp.maximum((y_ref[...] - m) * lax.rsqrt(var + 1e-5) * g_ref[...]
                    + bb_ref[...], 0.0)
    h_ref[...] = h
    a_ref[...] = _dot(wa_ref[...], h)
    b_ref[...] = _dot(wb_ref[...], h)


def _avg_body(c_ref, o_ref):
    o_ref[0, 0] = jnp.sum(jnp.log(c_ref[...] + 1.0)) * (1.0 / N)


def _post_body(h_ref, a_ref, sq_ref, sq2_ref, mn_ref, mx_ref, cnt_ref,
               avg_ref, wf_ref, bf_ref, y_ref, s_out, acc):
    i = pl.program_id(0)
    cntr = cnt_ref[0]
    cntc = jnp.maximum(cntr, 1.0)
    has = cntr > 0.0
    al = avg_ref[0, 0]
    amp = jnp.log(cntr + 1.0) * (1.0 / al)
    att = al / jnp.log(cntc + 1.0)
    A = a_ref[...]
    Sq = sq_ref[...]
    Sq2 = sq2_ref[...]
    mean = (cntr * A + Sq) / cntc
    sqm = (cntr * A * A + 2.0 * A * Sq + Sq2) / cntc
    std = jnp.sqrt(jnp.maximum(sqm - mean * mean, 0.0) + 1e-5)
    mn = jnp.where(has, A + mn_ref[...], 0.0)
    mx = jnp.where(has, A + mx_ref[...], 0.0)
    agg = jnp.concatenate([mean, mn, mx, std], axis=0)
    stacked = jnp.concatenate([h_ref[...], agg, agg * amp, agg * att], axis=0)
    y = _dot(wf_ref[...], stacked) + bf_ref[...]
    y_ref[...] = y
    col = lax.broadcasted_iota(jnp.int32, (1, NB2), 1) + i * NB2
    ym = jnp.where(col < N, y, 0.0)

    @pl.when(i == 0)
    def _():
        acc[...] = jnp.zeros((FP, 128), jnp.float32)

    acc[:, 0:1] += jnp.sum(ym, axis=1, keepdims=True)
    acc[:, 1:2] += jnp.sum(ym * ym, axis=1, keepdims=True)

    @pl.when(i == pl.num_programs(0) - 1)
    def _():
        s_out[...] = acc[...]


def _pool_body(y_ref, s_ref, g_ref, bb_ref, batch_ref,
               w1, b1, w2, b2, w3, b3, w4, b4, o_ref, acc):
    i = pl.program_id(0)
    m = s_ref[:, 0:1] * (1.0 / N)
    var = s_ref[:, 1:2] * (1.0 / N) - m * m
    h = jnp.maximum((y_ref[...] - m) * lax.rsqrt(var + 1e-5) * g_ref[...]
                    + bb_ref[...], 0.0)
    bb = batch_ref[0]
    gi = lax.broadcasted_iota(jnp.int32, (128, 1), 0)
    oh = (gi == bb).astype(jnp.float32)
    part = lax.dot_general(h, oh, (((1,), (1,)), ((), ())),
                           preferred_element_type=jnp.float32,
                           precision=lax.Precision.HIGHEST)

    @pl.when(i == 0)
    def _():
        acc[...] = jnp.zeros((FP, 128), jnp.float32)

    acc[...] += part

    @pl.when(i == pl.num_programs(0) - 1)
    def _():
        gt = acc[...]
        z = jnp.maximum(_dot(w1[...], gt) + b1[...], 0.0)
        z = jnp.maximum(_dot(w2[...], z) + b2[...], 0.0)
        z = jnp.maximum(_dot(w3[...], z) + b3[...], 0.0)
        o_ref[...] = _dot(w4[...], z) + b4[...]


_I0 = np.int32(0)


def _full(shape):
    return pl.BlockSpec(shape, lambda i: tuple(_I0 for _ in shape))


def _nodes(rows, nb):
    return pl.BlockSpec((rows, nb), lambda i: (_I0, i))


def _build_tc(interpret=False):
    f32 = jnp.float32
    emb = pl.pallas_call(
        _emb_body, grid=(NP // NB,), interpret=interpret,
        in_specs=[_nodes(8, NB), _full((FP, 8)), _full((FP, 1))],
        out_specs=_nodes(FP, NB),
        out_shape=jax.ShapeDtypeStruct((FP, NP), f32))
    ab_first = pl.pallas_call(
        _ab_first_body, grid=(NP // NB,), interpret=interpret,
        in_specs=[_nodes(FP, NB), _full((FP, FP)), _full((FP, FP))],
        out_specs=[_nodes(FP, NB), _nodes(FP, NB)],
        out_shape=[jax.ShapeDtypeStruct((FP, NP), f32)] * 2)
    ab_bn = pl.pallas_call(
        _ab_bn_body, grid=(NP // NB,), interpret=interpret,
        in_specs=[_nodes(FP, NB), _full((FP, 128)), _full((FP, 1)),
                  _full((FP, 1)), _full((FP, FP)), _full((FP, FP))],
        out_specs=[_nodes(FP, NB)] * 3,
        out_shape=[jax.ShapeDtypeStruct((FP, NP), f32)] * 3)
    avg = pl.pallas_call(
        _avg_body, interpret=interpret,
        in_specs=[pl.BlockSpec((NP // 128, 128), lambda: (_I0, _I0))],
        out_specs=pl.BlockSpec(memory_space=pltpu.SMEM),
        out_shape=jax.ShapeDtypeStruct((1, 1), f32))
    post = pl.pallas_call(
        _post_body, grid=(NP // NB2,), interpret=interpret,
        in_specs=[_nodes(FP, NB2), _nodes(FP, NB2), _nodes(FP, NB2),
                  _nodes(FP, NB2), _nodes(FP, NB2), _nodes(FP, NB2),
                  pl.BlockSpec((1, 1, NB2), lambda i: (i, _I0, _I0)),
                  _full((1, 1)), _full((FP, 13 * FP)), _full((FP, 1))],
        out_specs=[_nodes(FP, NB2), _full((FP, 128))],
        out_shape=[jax.ShapeDtypeStruct((FP, NP), f32),
                   jax.ShapeDtypeStruct((FP, 128), f32)],
        scratch_shapes=[pltpu.VMEM((FP, 128), f32)])
    pool = pl.pallas_call(
        _pool_body, grid=(NP // NB,), interpret=interpret,
        in_specs=[_nodes(FP, NB), _full((FP, 128)), _full((FP, 1)),
                  _full((FP, 1)),
                  pl.BlockSpec((1, 1, NB), lambda i: (i, _I0, _I0))]
                 + [_full((FP, FP)), _full((FP, 1))] * 4,
        out_specs=_full((FP, 128)),
        out_shape=jax.ShapeDtypeStruct((FP, 128), f32),
        scratch_shapes=[pltpu.VMEM((FP, 128), f32)])
    return emb, ab_first, ab_bn, avg, post, pool


_emb, _ab_first, _ab_bn, _avg, _post, _pool = _build_tc()


def _padt(w):
    return jnp.zeros((FP, FP), jnp.float32).at[:w.shape[1], :w.shape[0]].set(w.T)


def _padc(b, rows=FP):
    return jnp.zeros((rows, 1), jnp.float32).at[:b.shape[0], 0].set(b)


def _forward(x, edge_index, edge_attr, batch, params, tc, sc):
    _emb, _ab_first, _ab_bn, _avg, _post, _pool = tc
    src = edge_index[0].astype(jnp.int32)
    dst = edge_index[1].astype(jnp.int32)
    attr = edge_attr.astype(jnp.int32)

    xt = jnp.zeros((8, NP), jnp.float32).at[:5, :N].set(x.T)
    batch3 = (jnp.full((NP,), 127, jnp.int32).at[:N].set(batch.astype(jnp.int32))
              .reshape(NP // NB, 1, NB))

    embw = (jnp.zeros((FP, 8), jnp.float32)
            .at[:F, :5].set(params["emb1_W"].T))
    h = _emb(xt, embw, _padc(params["emb1_b"]))

    y = bns = cnt3 = avg = None
    for l, c in enumerate(params["convs"]):
        wat = _padt(c["Wpre"][:F])
        wbt = _padt(c["Wpre"][F:2 * F])
        ctab = ((params["edg_emb"] @ c["We"] + c["be"]) @ c["Wpre"][2 * F:]
                + c["bpre"])
        ctabt = jnp.zeros((FP, 16), jnp.float32).at[:F, :10].set(ctab.T)
        wf = c["Wpost"] @ c["Wlin"]
        wft = jnp.concatenate(
            [_padt(wf[k * F:(k + 1) * F]) for k in range(13)], axis=1)
        bf = _padc(c["bpost"] @ c["Wlin"] + c["blin"])

        if l == 0:
            at, bt = _ab_first(h, wat, wbt)
        else:
            cp = params["convs"][l - 1]
            h, at, bt = _ab_bn(y, bns, _padc(cp["bn_g"]), _padc(cp["bn_b"]),
                               wat, wbt)
        sq, sq2, mn, mx, cnt = sc(bt, ctabt, src, dst, attr)
        if l == 0:
            avg = _avg(cnt.reshape(NP // 128, 128))
            cnt3 = cnt.reshape(NP // NB2, 1, NB2)
        y, bns = _post(h, at, sq, sq2, mn, mx, cnt3, avg, wft, bf)

    c3 = params["convs"][3]
    (w1, b1), (w2, b2), (w3, b3), (w4, b4) = params["mlp"]
    out = _pool(y, bns, _padc(c3["bn_g"]), _padc(c3["bn_b"]), batch3,
                _padt(w1), _padc(b1), _padt(w2), _padc(b2),
                _padt(w3), _padc(b3), _padt(w4), _padc(b4))
    return out[0, :NG][:, None]


def kernel(x, edge_index, edge_attr, batch, params):
    return _forward(x, edge_index, edge_attr, batch, params,
                    (_emb, _ab_first, _ab_bn, _avg, _post, _pool),
                    lambda *a: _get_sc_edge()(*a))


# speedup vs baseline: 2.4330x; 1.2242x over previous
import functools

import numpy as np

import jax
import jax.numpy as jnp
from jax import lax
from jax.experimental import pallas as pl
from jax.experimental.pallas import tpu as pltpu
from jax.experimental.pallas import tpu_sc as plsc

F = 126
FP = 128
N = 10000
NP = 10240
E = 160000
NG = 64

NC = 2
NS = 16
NW = NC * NS
EW = 1600
NWIN = E // EW
GPW = EW // 16
GUNROLL = 5
HMASK = 4095
NPT = NP // NW
FLT_MAX = 3.4028235e38

NB = 2048
NB2 = 1024


def _sc_edge_body(bt, ctab, srch, dsth, attrh,
                  sq_o, sq2_o, mn_o, mx_o, cnt_o,
                  brow0, brow1, ct0, ct1,
                  srcA, dstA, attrA, srcB, dstB, attrB, semA, semB,
                  tmpw, cntacc,
                  asq0, asq20, amn0, amx0, asq1, asq21, amn1, amx1):
    wid = lax.axis_index("s") * NC + lax.axis_index("c")
    lanes = lax.broadcasted_iota(jnp.int32, (16,), 0)
    ones = jnp.full((16,), 1.0, jnp.float32)
    zero16 = jnp.zeros((16,), jnp.float32)
    big16 = jnp.full((16,), FLT_MAX, jnp.float32)
    hmask = jnp.full((16,), HMASK, jnp.int32)

    def start_fetch(w, sw, dw, aw, sem):
        off = w * EW
        pltpu.make_async_copy(srch.at[pl.ds(off, EW)], sw, sem).start()
        pltpu.make_async_copy(dsth.at[pl.ds(off, EW)], dw, sem).start()
        pltpu.make_async_copy(attrh.at[pl.ds(off, EW)], aw, sem).start()

    def wait_fetch(w, sw, dw, aw, sem):
        off = w * EW
        pltpu.make_async_copy(srch.at[pl.ds(off, EW)], sw, sem).wait()
        pltpu.make_async_copy(dsth.at[pl.ds(off, EW)], dw, sem).wait()
        pltpu.make_async_copy(attrh.at[pl.ds(off, EW)], aw, sem).wait()

    for p in range(2):
        f0 = p * 64 + wid * 2
        pltpu.sync_copy(bt.at[f0], brow0)
        pltpu.sync_copy(bt.at[f0 + 1], brow1)
        pltpu.sync_copy(ctab.at[f0], ct0)
        pltpu.sync_copy(ctab.at[f0 + 1], ct1)

        def init_body(k, _, p=p):
            sl = pl.ds(k * 16, 16)
            asq0[sl] = zero16
            asq20[sl] = zero16
            amn0[sl] = big16
            amx0[sl] = -big16
            asq1[sl] = zero16
            asq21[sl] = zero16
            amn1[sl] = big16
            amx1[sl] = -big16
            if p == 0:
                cntacc[sl] = zero16
            return jnp.int32(0)

        lax.fori_loop(jnp.int32(0), jnp.int32(NP // 16), init_body, jnp.int32(0))

        def mm_round(q0, q1, dstv, hv, remb):
            plsc.store_scatter(tmpw, [hv], lanes, mask=remb)
            r = plsc.load_gather(tmpw, [hv], mask=remb)
            if remb is None:
                win = r == lanes
                left = r != lanes
            else:
                win = jnp.logical_and(remb, r == lanes)
                left = jnp.logical_and(remb, jnp.logical_not(win))
            for q, amn, amx in ((q0, amn0, amx0), (q1, amn1, amx1)):
                mv = plsc.load_gather(amn, [dstv], mask=win)
                plsc.store_scatter(amn, [dstv], jnp.minimum(mv, q), mask=win)
                Mv = plsc.load_gather(amx, [dstv], mask=win)
                plsc.store_scatter(amx, [dstv], jnp.maximum(Mv, q), mask=win)
            return left

        def process(srcw, dstw, attrw, p=p):
            def grp_one(g):
                sl = pl.ds(g * 16, 16)
                srcv = srcw[sl]
                dstv = dstw[sl]
                attrv = attrw[sl]
                if p == 0:
                    plsc.addupdate_scatter(cntacc, [dstv], ones)
                qs = []
                for brow, ct, asq, asq2 in (
                        (brow0, ct0, asq0, asq20),
                        (brow1, ct1, asq1, asq21)):
                    b = plsc.load_gather(brow, [srcv])
                    cg = plsc.load_gather(ct, [attrv])
                    q = b + cg
                    plsc.addupdate_scatter(asq, [dstv], q)
                    plsc.addupdate_scatter(asq2, [dstv], q * q)
                    qs.append(q)
                hv = jnp.bitwise_and(dstv, hmask)
                left = mm_round(qs[0], qs[1], dstv, hv, None)
                nleft = plsc.all_reduce_population_count(left)[0]

                @pl.when(nleft > 0)
                def _():
                    def cond(carry):
                        return carry[1] > 0

                    def wbody(carry):
                        rem_i, _ = carry
                        l2 = mm_round(qs[0], qs[1], dstv, hv, rem_i > 0)
                        return (l2.astype(jnp.int32),
                                plsc.all_reduce_population_count(l2)[0])

                    lax.while_loop(cond, wbody,
                                   (left.astype(jnp.int32), nleft))

            def grp_body(gg, _):
                g0 = gg * jnp.int32(GUNROLL)
                for u in range(GUNROLL):
                    grp_one(g0 + jnp.int32(u))
                return jnp.int32(0)

            lax.fori_loop(jnp.int32(0), jnp.int32(GPW // GUNROLL), grp_body,
                          jnp.int32(0))

        start_fetch(jnp.int32(0), srcA, dstA, attrA, semA)

        def win_pair(wp, _, p=p):
            w0 = wp * jnp.int32(2)
            start_fetch(w0 + 1, srcB, dstB, attrB, semB)
            wait_fetch(w0, srcA, dstA, attrA, semA)
            process(srcA, dstA, attrA, p=p)

            @pl.when(w0 + 2 < NWIN)
            def _():
                start_fetch(w0 + 2, srcA, dstA, attrA, semA)

            wait_fetch(w0 + 1, srcB, dstB, attrB, semB)
            process(srcB, dstB, attrB, p=p)
            return jnp.int32(0)

        lax.fori_loop(jnp.int32(0), jnp.int32(NWIN // 2), win_pair,
                      jnp.int32(0))

        pltpu.sync_copy(asq0, sq_o.at[f0])
        pltpu.sync_copy(asq1, sq_o.at[f0 + 1])
        pltpu.sync_copy(asq20, sq2_o.at[f0])
        pltpu.sync_copy(asq21, sq2_o.at[f0 + 1])
        pltpu.sync_copy(amn0, mn_o.at[f0])
        pltpu.sync_copy(amn1, mn_o.at[f0 + 1])
        pltpu.sync_copy(amx0, mx_o.at[f0])
        pltpu.sync_copy(amx1, mx_o.at[f0 + 1])
        if p == 0:
            s0 = wid * NPT
            pltpu.sync_copy(cntacc.at[pl.ds(s0, NPT)], cnt_o.at[pl.ds(s0, NPT)])


@functools.cache
def _get_sc_edge():
    return pl.kernel(
        _sc_edge_body,
        out_type=[jax.ShapeDtypeStruct((FP, NP), jnp.float32),
                  jax.ShapeDtypeStruct((FP, NP), jnp.float32),
                  jax.ShapeDtypeStruct((FP, NP), jnp.float32),
                  jax.ShapeDtypeStruct((FP, NP), jnp.float32),
                  jax.ShapeDtypeStruct((NP,), jnp.float32)],
        mesh=plsc.VectorSubcoreMesh(core_axis_name="c", subcore_axis_name="s",
                                    num_cores=NC, num_subcores=NS),
        compiler_params=pltpu.CompilerParams(needs_layout_passes=False),
        scratch_types=[pltpu.VMEM((NP,), jnp.float32),
                       pltpu.VMEM((NP,), jnp.float32),
                       pltpu.VMEM((16,), jnp.float32),
                       pltpu.VMEM((16,), jnp.float32),
                       pltpu.VMEM((EW,), jnp.int32),
                       pltpu.VMEM((EW,), jnp.int32),
                       pltpu.VMEM((EW,), jnp.int32),
                       pltpu.VMEM((EW,), jnp.int32),
                       pltpu.VMEM((EW,), jnp.int32),
                       pltpu.VMEM((EW,), jnp.int32),
                       pltpu.SemaphoreType.DMA,
                       pltpu.SemaphoreType.DMA,
                       pltpu.VMEM((HMASK + 1,), jnp.int32),
                       pltpu.VMEM((NP,), jnp.float32),
                       pltpu.VMEM((NP,), jnp.float32),
                       pltpu.VMEM((NP,), jnp.float32),
                       pltpu.VMEM((NP,), jnp.float32),
                       pltpu.VMEM((NP,), jnp.float32),
                       pltpu.VMEM((NP,), jnp.float32),
                       pltpu.VMEM((NP,), jnp.float32),
                       pltpu.VMEM((NP,), jnp.float32),
                       pltpu.VMEM((NP,), jnp.float32)],
    )


def _dot(a, b):
    return jnp.dot(a, b, preferred_element_type=jnp.float32,
                   precision=lax.Precision.HIGHEST)


def _emb_body(xt_ref, w_ref, b_ref, o_ref):
    o_ref[...] = _dot(w_ref[...], xt_ref[...]) + b_ref[...]


def _ab_first_body(h_ref, wa_ref, wb_ref, a_ref, b_ref):
    h = h_ref[...]
    a_ref[...] = _dot(wa_ref[...], h)
    b_ref[...] = _dot(wb_ref[...], h)


def _ab_bn_body(y_ref, s_ref, g_ref, bb_ref, wa_ref, wb_ref,
                h_ref, a_ref, b_ref):
    m = s_ref[:, 0:1] * (1.0 / N)
    var = s_ref[:, 1:2] * (1.0 / N) - m * m
    h = jnp.maximum((y_ref[...] - m) * lax.rsqrt(var + 1e-5) * g_ref[...]
                    + bb_ref[...], 0.0)
    h_ref[...] = h
    a_ref[...] = _dot(wa_ref[...], h)
    b_ref[...] = _dot(wb_ref[...], h)


def _avg_body(c_ref, o_ref):
    o_ref[0, 0] = jnp.sum(jnp.log(c_ref[...] + 1.0)) * (1.0 / N)


def _post_body(h_ref, a_ref, sq_ref, sq2_ref, mn_ref, mx_ref, cnt_ref,
               avg_ref, wf_ref, bf_ref, y_ref, s_out, acc):
    i = pl.program_id(0)
    cntr = cnt_ref[0]
    cntc = jnp.maximum(cntr, 1.0)
    has = cntr > 0.0
    al = avg_ref[0, 0]
    amp = jnp.log(cntr + 1.0) * (1.0 / al)
    att = al / jnp.log(cntc + 1.0)
    A = a_ref[...]
    Sq = sq_ref[...]
    Sq2 = sq2_ref[...]
    mean = (cntr * A + Sq) / cntc
    sqm = (cntr * A * A + 2.0 * A * Sq + Sq2) / cntc
    std = jnp.sqrt(jnp.maximum(sqm - mean * mean, 0.0) + 1e-5)
    mn = jnp.where(has, A + mn_ref[...], 0.0)
    mx = jnp.where(has, A + mx_ref[...], 0.0)
    agg = jnp.concatenate([mean, mn, mx, std], axis=0)
    stacked = jnp.concatenate([h_ref[...], agg, agg * amp, agg * att], axis=0)
    y = _dot(wf_ref[...], stacked) + bf_ref[...]
    y_ref[...] = y
    col = lax.broadcasted_iota(jnp.int32, (1, NB2), 1) + i * NB2
    ym = jnp.where(col < N, y, 0.0)

    @pl.when(i == 0)
    def _():
        acc[...] = jnp.zeros((FP, 128), jnp.float32)

    acc[:, 0:1] += jnp.sum(ym, axis=1, keepdims=True)
    acc[:, 1:2] += jnp.sum(ym * ym, axis=1, keepdims=True)

    @pl.when(i == pl.num_programs(0) - 1)
    def _():
        s_out[...] = acc[...]


def _pool_body(y_ref, s_ref, g_ref, bb_ref, batch_ref,
               w1, b1, w2, b2, w3, b3, w4, b4, o_ref, acc):
    i = pl.program_id(0)
    m = s_ref[:, 0:1] * (1.0 / N)
    var = s_ref[:, 1:2] * (1.0 / N) - m * m
    h = jnp.maximum((y_ref[...] - m) * lax.rsqrt(var + 1e-5) * g_ref[...]
                    + bb_ref[...], 0.0)
    bb = batch_ref[0]
    gi = lax.broadcasted_iota(jnp.int32, (128, 1), 0)
    oh = (gi == bb).astype(jnp.float32)
    part = lax.dot_general(h, oh, (((1,), (1,)), ((), ())),
                           preferred_element_type=jnp.float32,
                           precision=lax.Precision.HIGHEST)

    @pl.when(i == 0)
    def _():
        acc[...] = jnp.zeros((FP, 128), jnp.float32)

    acc[...] += part

    @pl.when(i == pl.num_programs(0) - 1)
    def _():
        gt = acc[...]
        z = jnp.maximum(_dot(w1[...], gt) + b1[...], 0.0)
        z = jnp.maximum(_dot(w2[...], z) + b2[...], 0.0)
        z = jnp.maximum(_dot(w3[...], z) + b3[...], 0.0)
        o_ref[...] = _dot(w4[...], z) + b4[...]


_I0 = np.int32(0)


def _full(shape):
    return pl.BlockSpec(shape, lambda i: tuple(_I0 for _ in shape))


def _nodes(rows, nb):
    return pl.BlockSpec((rows, nb), lambda i: (_I0, i))


def _build_tc(interpret=False):
    f32 = jnp.float32
    emb = pl.pallas_call(
        _emb_body, grid=(NP // NB,), interpret=interpret,
        in_specs=[_nodes(8, NB), _full((FP, 8)), _full((FP, 1))],
        out_specs=_nodes(FP, NB),
        out_shape=jax.ShapeDtypeStruct((FP, NP), f32))
    ab_first = pl.pallas_call(
        _ab_first_body, grid=(NP // NB,), interpret=interpret,
        in_specs=[_nodes(FP, NB), _full((FP, FP)), _full((FP, FP))],
        out_specs=[_nodes(FP, NB), _nodes(FP, NB)],
        out_shape=[jax.ShapeDtypeStruct((FP, NP), f32)] * 2)
    ab_bn = pl.pallas_call(
        _ab_bn_body, grid=(NP // NB,), interpret=interpret,
        in_specs=[_nodes(FP, NB), _full((FP, 128)), _full((FP, 1)),
                  _full((FP, 1)), _full((FP, FP)), _full((FP, FP))],
        out_specs=[_nodes(FP, NB)] * 3,
        out_shape=[jax.ShapeDtypeStruct((FP, NP), f32)] * 3)
    avg = pl.pallas_call(
        _avg_body, interpret=interpret,
        in_specs=[pl.BlockSpec((NP // 128, 128), lambda: (_I0, _I0))],
        out_specs=pl.BlockSpec(memory_space=pltpu.SMEM),
        out_shape=jax.ShapeDtypeStruct((1, 1), f32))
    post = pl.pallas_call(
        _post_body, grid=(NP // NB2,), interpret=interpret,
        in_specs=[_nodes(FP, NB2), _nodes(FP, NB2), _nodes(FP, NB2),
                  _nodes(FP, NB2), _nodes(FP, NB2), _nodes(FP, NB2),
                  pl.BlockSpec((1, 1, NB2), lambda i: (i, _I0, _I0)),
                  _full((1, 1)), _full((FP, 13 * FP)), _full((FP, 1))],
        out_specs=[_nodes(FP, NB2), _full((FP, 128))],
        out_shape=[jax.ShapeDtypeStruct((FP, NP), f32),
                   jax.ShapeDtypeStruct((FP, 128), f32)],
        scratch_shapes=[pltpu.VMEM((FP, 128), f32)])
    pool = pl.pallas_call(
        _pool_body, grid=(NP // NB,), interpret=interpret,
        in_specs=[_nodes(FP, NB), _full((FP, 128)), _full((FP, 1)),
                  _full((FP, 1)),
                  pl.BlockSpec((1, 1, NB), lambda i: (i, _I0, _I0))]
                 + [_full((FP, FP)), _full((FP, 1))] * 4,
        out_specs=_full((FP, 128)),
        out_shape=jax.ShapeDtypeStruct((FP, 128), f32),
        scratch_shapes=[pltpu.VMEM((FP, 128), f32)])
    return emb, ab_first, ab_bn, avg, post, pool


_emb, _ab_first, _ab_bn, _avg, _post, _pool = _build_tc()


def _padt(w):
    return jnp.zeros((FP, FP), jnp.float32).at[:w.shape[1], :w.shape[0]].set(w.T)


def _padc(b, rows=FP):
    return jnp.zeros((rows, 1), jnp.float32).at[:b.shape[0], 0].set(b)


def _forward(x, edge_index, edge_attr, batch, params, tc, sc):
    _emb, _ab_first, _ab_bn, _avg, _post, _pool = tc
    src = edge_index[0].astype(jnp.int32)
    dst = edge_index[1].astype(jnp.int32)
    attr = edge_attr.astype(jnp.int32)

    xt = jnp.zeros((8, NP), jnp.float32).at[:5, :N].set(x.T)
    batch3 = (jnp.full((NP,), 127, jnp.int32).at[:N].set(batch.astype(jnp.int32))
              .reshape(NP // NB, 1, NB))

    embw = (jnp.zeros((FP, 8), jnp.float32)
            .at[:F, :5].set(params["emb1_W"].T))
    h = _emb(xt, embw, _padc(params["emb1_b"]))

    y = bns = cnt3 = avg = None
    for l, c in enumerate(params["convs"]):
        wat = _padt(c["Wpre"][:F])
        wbt = _padt(c["Wpre"][F:2 * F])
        ctab = ((params["edg_emb"] @ c["We"] + c["be"]) @ c["Wpre"][2 * F:]
                + c["bpre"])
        ctabt = jnp.zeros((FP, 16), jnp.float32).at[:F, :10].set(ctab.T)
        wf = c["Wpost"] @ c["Wlin"]
        wft = jnp.concatenate(
            [_padt(wf[k * F:(k + 1) * F]) for k in range(13)], axis=1)
        bf = _padc(c["bpost"] @ c["Wlin"] + c["blin"])

        if l == 0:
            at, bt = _ab_first(h, wat, wbt)
        else:
            cp = params["convs"][l - 1]
            h, at, bt = _ab_bn(y, bns, _padc(cp["bn_g"]), _padc(cp["bn_b"]),
                               wat, wbt)
        sq, sq2, mn, mx, cnt = sc(bt, ctabt, src, dst, attr)
        if l == 0:
            avg = _avg(cnt.reshape(NP // 128, 128))
            cnt3 = cnt.reshape(NP // NB2, 1, NB2)
        y, bns = _post(h, at, sq, sq2, mn, mx, cnt3, avg, wft, bf)

    c3 = params["convs"][3]
    (w1, b1), (w2, b2), (w3, b3), (w4, b4) = params["mlp"]
    out = _pool(y, bns, _padc(c3["bn_g"]), _padc(c3["bn_b"]), batch3,
                _padt(w1), _padc(b1), _padt(w2), _padc(b2),
                _padt(w3), _padc(b3), _padt(w4), _padc(b4))
    return out[0, :NG][:, None]


def kernel(x, edge_index, edge_attr, batch, params):
    return _forward(x, edge_index, edge_attr, batch, params,
                    (_emb, _ab_first, _ab_bn, _avg, _post, _pool),
                    lambda *a: _get_sc_edge()(*a))
